# baseline (speedup 1.0000x reference)
import functools

import jax
import jax.numpy as jnp
from jax import lax
from jax.experimental import pallas as pl
from jax.experimental.pallas import tpu as pltpu
from jax.experimental.pallas import tpu_sc as plsc

OUT_DIM = 4096
N_BITS = 2048
LANES = 16


@functools.partial(
    pl.kernel,
    out_type=jax.ShapeDtypeStruct((OUT_DIM,), jnp.float32),
    mesh=plsc.VectorSubcoreMesh(core_axis_name="c", subcore_axis_name="s"),
    scratch_types=[
        pltpu.VMEM((N_BITS,), jnp.int32),
        pltpu.VMEM((N_BITS,), jnp.int32),
        pltpu.VMEM((OUT_DIM,), jnp.float32),
    ],
    compiler_params=pltpu.CompilerParams(needs_layout_passes=False),
)
def _scatter_row(base_hbm, pos_hbm, sign_hbm, out_hbm, pos_v, sign_v, row_v):
    cid = lax.axis_index("c")
    sid = lax.axis_index("s")

    @pl.when(jnp.logical_and(cid == 0, sid == 0))
    def _():
        pltpu.sync_copy(base_hbm, row_v)
        pltpu.sync_copy(pos_hbm, pos_v)
        pltpu.sync_copy(sign_hbm, sign_v)

        ones = jnp.full((LANES,), 1.0, jnp.float32)
        neg_ones = jnp.full((LANES,), -1.0, jnp.float32)

        def pos_pass(i, carry):
            p = pos_v[pl.ds(i * LANES, LANES)]
            s = sign_v[pl.ds(i * LANES, LANES)]
            plsc.store_scatter(row_v, [p], ones, mask=s == 1)
            return carry

        lax.fori_loop(0, N_BITS // LANES, pos_pass, 0)

        def neg_pass(i, carry):
            p = pos_v[pl.ds(i * LANES, LANES)]
            s = sign_v[pl.ds(i * LANES, LANES)]
            plsc.store_scatter(row_v, [p], neg_ones, mask=s != 1)
            return carry

        lax.fori_loop(0, N_BITS // LANES, neg_pass, 0)

        pltpu.sync_copy(row_v, out_hbm)


def kernel(on_bits, n_cols):
    on_bits = on_bits.astype(jnp.int32)
    positions = on_bits[:, 0]
    signs = on_bits[:, 1]
    col = jnp.arange(OUT_DIM, dtype=jnp.int32)
    base = jnp.where(col < n_cols, 0.0, -10.0).astype(jnp.float32)
    h = _scatter_row(base, positions, signs)[None, :]
    return (h, h)


# speedup vs baseline: 1.0678x; 1.0678x over previous
import functools

import jax
import jax.numpy as jnp
from jax import lax
from jax.experimental import pallas as pl
from jax.experimental.pallas import tpu as pltpu
from jax.experimental.pallas import tpu_sc as plsc

OUT_DIM = 4096
N_BITS = 2048
LANES = 16
UNROLL = 4
N_CHUNKS = N_BITS // LANES


@functools.partial(
    pl.kernel,
    out_type=(
        jax.ShapeDtypeStruct((OUT_DIM,), jnp.float32),
        jax.ShapeDtypeStruct((OUT_DIM,), jnp.float32),
    ),
    mesh=plsc.VectorSubcoreMesh(core_axis_name="c", subcore_axis_name="s"),
    scratch_types=[
        pltpu.VMEM((2 * N_BITS,), jnp.int32),
        pltpu.VMEM((OUT_DIM,), jnp.float32),
    ],
    compiler_params=pltpu.CompilerParams(needs_layout_passes=False),
)
def _scatter_row(base_hbm, ob_hbm, out0_hbm, out1_hbm, ob_v, row_v):
    cid = lax.axis_index("c")
    sid = lax.axis_index("s")

    @pl.when(jnp.logical_and(cid == 0, sid == 0))
    def _():
        pltpu.sync_copy(base_hbm, row_v)
        pltpu.sync_copy(ob_hbm, ob_v)

        ones = jnp.full((LANES,), 1.0, jnp.float32)
        neg_ones = jnp.full((LANES,), -1.0, jnp.float32)
        pos_ids = lax.iota(jnp.int32, LANES) * 2
        sign_ids = pos_ids + 1

        def chunk(k, value, mask_pos):
            p = plsc.load_gather(ob_v, [pos_ids + 2 * LANES * k])
            s = plsc.load_gather(ob_v, [sign_ids + 2 * LANES * k])
            m = (s == 1) if mask_pos else (s != 1)
            plsc.store_scatter(row_v, [p], value, mask=m)

        def pos_pass(i, carry):
            for j in range(UNROLL):
                chunk(i * UNROLL + j, ones, True)
            return carry

        lax.fori_loop(0, N_CHUNKS // UNROLL, pos_pass, 0)

        def neg_pass(i, carry):
            for j in range(UNROLL):
                chunk(i * UNROLL + j, neg_ones, False)
            return carry

        lax.fori_loop(0, N_CHUNKS // UNROLL, neg_pass, 0)

        pltpu.sync_copy(row_v, out0_hbm)
        pltpu.sync_copy(row_v, out1_hbm)


def kernel(on_bits, n_cols):
    ob_flat = on_bits.astype(jnp.int32).reshape(2 * N_BITS)
    col = jnp.arange(OUT_DIM, dtype=jnp.int32)
    base = jnp.where(col < n_cols, 0.0, -10.0).astype(jnp.float32)
    h0, h1 = _scatter_row(base, ob_flat)
    return (h0[None, :], h1[None, :])


# speedup vs baseline: 1.1086x; 1.0382x over previous
import functools

import jax
import jax.numpy as jnp
from jax import lax
from jax.experimental import pallas as pl
from jax.experimental.pallas import tpu as pltpu
from jax.experimental.pallas import tpu_sc as plsc

OUT_DIM = 4096
N_BITS = 2048
LANES = 16
N_CHUNKS = N_BITS // LANES


@functools.partial(
    pl.kernel,
    out_type=(
        jax.ShapeDtypeStruct((OUT_DIM,), jnp.float32),
        jax.ShapeDtypeStruct((OUT_DIM,), jnp.float32),
    ),
    mesh=plsc.VectorSubcoreMesh(core_axis_name="c", subcore_axis_name="s"),
    scratch_types=[
        pltpu.VMEM((N_BITS,), jnp.int32),
        pltpu.VMEM((N_BITS,), jnp.int32),
        pltpu.VMEM((OUT_DIM,), jnp.float32),
        pltpu.SemaphoreType.DMA,
        pltpu.SemaphoreType.DMA,
        pltpu.SemaphoreType.DMA,
    ],
    compiler_params=pltpu.CompilerParams(needs_layout_passes=False),
)
def _scatter_row(base_hbm, pos_hbm, sign_hbm, out0_hbm, out1_hbm,
                 pos_v, sign_v, row_v, sem0, sem1, sem2):
    cid = lax.axis_index("c")
    sid = lax.axis_index("s")

    @pl.when(jnp.logical_and(cid == 0, sid == 0))
    def _():
        c0 = pltpu.async_copy(base_hbm, row_v, sem0)
        c1 = pltpu.async_copy(pos_hbm, pos_v, sem1)
        c2 = pltpu.async_copy(sign_hbm, sign_v, sem2)
        c0.wait()
        c1.wait()
        c2.wait()

        def body(i, carry):
            neg = i >= N_CHUNKS
            k = i & (N_CHUNKS - 1)
            p = pos_v[pl.ds(k * LANES, LANES)]
            s = sign_v[pl.ds(k * LANES, LANES)]
            val = jnp.broadcast_to(
                jnp.where(neg, jnp.float32(-1.0), jnp.float32(1.0)), (LANES,))
            m = (s == 1) != neg
            plsc.store_scatter(row_v, [p], val, mask=m)
            return carry

        lax.fori_loop(0, 2 * N_CHUNKS, body, 0)

        d0 = pltpu.async_copy(row_v, out0_hbm, sem0)
        d1 = pltpu.async_copy(row_v, out1_hbm, sem1)
        d0.wait()
        d1.wait()


def kernel(on_bits, n_cols):
    on_bits = on_bits.astype(jnp.int32)
    positions = on_bits[:, 0]
    signs = on_bits[:, 1]
    col = jnp.arange(OUT_DIM, dtype=jnp.int32)
    base = jnp.where(col < n_cols, 0.0, -10.0).astype(jnp.float32)
    h0, h1 = _scatter_row(base, positions, signs)
    return (h0[None, :], h1[None, :])


# speedup vs baseline: 1.1856x; 1.0694x over previous
import functools

import jax
import jax.numpy as jnp
from jax import lax
from jax.experimental import pallas as pl
from jax.experimental.pallas import tpu as pltpu
from jax.experimental.pallas import tpu_sc as plsc

OUT_DIM = 4096
N_BITS = 2048
LANES = 16
N_CHUNKS = N_BITS // LANES


@functools.partial(
    pl.kernel,
    out_type=(
        jax.ShapeDtypeStruct((OUT_DIM,), jnp.float32),
        jax.ShapeDtypeStruct((OUT_DIM,), jnp.float32),
    ),
    mesh=plsc.VectorSubcoreMesh(core_axis_name="c", subcore_axis_name="s",
                                num_cores=1),
    scratch_types=[
        pltpu.VMEM((N_BITS,), jnp.int32),
        pltpu.VMEM((N_BITS,), jnp.int32),
        pltpu.VMEM((OUT_DIM,), jnp.float32),
        pltpu.SemaphoreType.DMA,
        pltpu.SemaphoreType.DMA,
        pltpu.SemaphoreType.DMA,
    ],
    compiler_params=pltpu.CompilerParams(needs_layout_passes=False),
)
def _scatter_row(base_hbm, pos_hbm, sign_hbm, out0_hbm, out1_hbm,
                 pos_v, sign_v, row_v, sem0, sem1, sem2):
    cid = lax.axis_index("c")
    sid = lax.axis_index("s")

    @pl.when(jnp.logical_and(cid == 0, sid == 0))
    def _():
        c0 = pltpu.async_copy(base_hbm, row_v, sem0)
        c1 = pltpu.async_copy(pos_hbm, pos_v, sem1)
        c2 = pltpu.async_copy(sign_hbm, sign_v, sem2)
        c0.wait()
        c1.wait()
        c2.wait()

        ones = jnp.full((LANES,), 1.0, jnp.float32)
        neg_ones = jnp.full((LANES,), -1.0, jnp.float32)

        def pos_pass(i, carry):
            p = pos_v[pl.ds(i * LANES, LANES)]
            s = sign_v[pl.ds(i * LANES, LANES)]
            plsc.store_scatter(row_v, [p], ones, mask=s == 1)
            return carry

        lax.fori_loop(0, N_CHUNKS, pos_pass, 0)

        def neg_pass(i, carry):
            p = pos_v[pl.ds(i * LANES, LANES)]
            s = sign_v[pl.ds(i * LANES, LANES)]
            plsc.store_scatter(row_v, [p], neg_ones, mask=s != 1)
            return carry

        lax.fori_loop(0, N_CHUNKS, neg_pass, 0)

        d0 = pltpu.async_copy(row_v, out0_hbm, sem0)
        d1 = pltpu.async_copy(row_v, out1_hbm, sem1)
        d0.wait()
        d1.wait()


def kernel(on_bits, n_cols):
    on_bits = on_bits.astype(jnp.int32)
    positions = on_bits[:, 0]
    signs = on_bits[:, 1]
    col = jnp.arange(OUT_DIM, dtype=jnp.int32)
    base = jnp.where(col < n_cols, 0.0, -10.0).astype(jnp.float32)
    h0, h1 = _scatter_row(base, positions, signs)
    return (h0[None, :], h1[None, :])


# speedup vs baseline: 1.2840x; 1.0830x over previous
import functools

import jax
import jax.numpy as jnp
from jax import lax
from jax.experimental import pallas as pl
from jax.experimental.pallas import tpu as pltpu
from jax.experimental.pallas import tpu_sc as plsc

OUT_DIM = 4096
N_BITS = 2048
LANES = 16
N_CHUNKS = N_BITS // LANES


@functools.partial(
    pl.kernel,
    out_type=(
        jax.ShapeDtypeStruct((OUT_DIM,), jnp.float32),
        jax.ShapeDtypeStruct((OUT_DIM,), jnp.float32),
    ),
    mesh=plsc.VectorSubcoreMesh(core_axis_name="c", subcore_axis_name="s",
                                num_cores=1),
    scratch_types=[
        pltpu.VMEM((N_BITS,), jnp.int32),
        pltpu.VMEM((N_BITS,), jnp.int32),
        pltpu.VMEM((OUT_DIM,), jnp.float32),
        pltpu.SemaphoreType.DMA,
        pltpu.SemaphoreType.DMA,
        pltpu.SemaphoreType.DMA,
    ],
    compiler_params=pltpu.CompilerParams(needs_layout_passes=False),
)
def _scatter_row(base_hbm, pos_hbm, sign_hbm, out0_hbm, out1_hbm,
                 pos_v, sign_v, row_v, sem0, sem1, sem2):
    cid = lax.axis_index("c")
    sid = lax.axis_index("s")

    @pl.when(jnp.logical_and(cid == 0, sid == 0))
    def _():
        c0 = pltpu.async_copy(base_hbm, row_v, sem0)
        c1 = pltpu.async_copy(pos_hbm, pos_v, sem1)
        c2 = pltpu.async_copy(sign_hbm, sign_v, sem2)
        c0.wait()
        c1.wait()
        c2.wait()

        ones = jnp.full((LANES,), 1.0, jnp.float32)
        neg_ones = jnp.full((LANES,), -1.0, jnp.float32)

        @plsc.parallel_loop(0, N_BITS, step=LANES, unroll=4)
        def pos_pass(i):
            p = pos_v[pl.ds(i, LANES)]
            s = sign_v[pl.ds(i, LANES)]
            plsc.store_scatter(row_v, [p], ones, mask=s == 1)

        @plsc.parallel_loop(0, N_BITS, step=LANES, unroll=4)
        def neg_pass(i):
            p = pos_v[pl.ds(i, LANES)]
            s = sign_v[pl.ds(i, LANES)]
            plsc.store_scatter(row_v, [p], neg_ones, mask=s != 1)

        d0 = pltpu.async_copy(row_v, out0_hbm, sem0)
        d1 = pltpu.async_copy(row_v, out1_hbm, sem1)
        d0.wait()
        d1.wait()


def kernel(on_bits, n_cols):
    on_bits = on_bits.astype(jnp.int32)
    positions = on_bits[:, 0]
    signs = on_bits[:, 1]
    col = jnp.arange(OUT_DIM, dtype=jnp.int32)
    base = jnp.where(col < n_cols, 0.0, -10.0).astype(jnp.float32)
    h0, h1 = _scatter_row(base, positions, signs)
    return (h0[None, :], h1[None, :])
